# Initial kernel scaffold; baseline (speedup 1.0000x reference)
import functools

import jax
import jax.numpy as jnp
from jax import lax
from jax.experimental import pallas as pl
from jax.experimental.pallas import tpu as pltpu
from jax.experimental.pallas import tpu_sc as plsc

D = 32
NC = 2
NS = 16
NW = NC * NS
CH = 128
NBUF = 4


@functools.partial(jax.jit, static_argnames=("n_ch", "b_per_w"))
def _go2vec_sc(idx, table, *, n_ch, b_per_w):
    B = NW * b_per_w
    mesh = plsc.VectorSubcoreMesh(core_axis_name="c", subcore_axis_name="s")

    @functools.partial(
        pl.kernel,
        mesh=mesh,
        out_type=jax.ShapeDtypeStruct((B, D), jnp.float32),
        scratch_types=[
            pltpu.VMEM((n_ch, CH), jnp.int32),
            pltpu.VMEM((NBUF, CH, D), jnp.float32),
            pltpu.SemaphoreType.DMA,
            pltpu.SemaphoreType.DMA,
        ],
    )
    def k(idx_hbm, table_hbm, out_hbm, idx_v, rows_v, gsem, osem):
        wid = lax.axis_index("s") * NC + lax.axis_index("c")
        base = wid * b_per_w

        pltpu.sync_copy(idx_hbm.at[wid], idx_v)

        def g_copy(j, p):
            return pltpu.make_async_copy(
                table_hbm.at[idx_v.at[j]], rows_v.at[p], gsem)

        def o_copy(j, p):
            return pltpu.make_async_copy(
                rows_v.at[p], out_hbm.at[pl.ds(base + j * CH, CH)], osem)

        g_copy(0, 0).start()

        def body(j, carry):
            p = lax.rem(j, NBUF)
            g_copy(j, p).wait()
            o_copy(j, p).start()

            @pl.when(j < n_ch - 1)
            def _():
                pn = lax.rem(j + 1, NBUF)

                @pl.when(j >= NBUF - 1)
                def _():
                    o_copy(j - (NBUF - 1), pn).wait()

                g_copy(j + 1, pn).start()

            return carry

        lax.fori_loop(0, n_ch, body, 0)

        for b in range(NBUF):
            o_copy(0, b).wait()

    return k(idx, table)


def kernel(go, emb_weights):
    B_, H_ = go.shape
    B = B_ * H_
    b_per_w = B // NW
    n_ch = b_per_w // CH
    idx = go.reshape(NW, n_ch, CH).astype(jnp.int32)
    out = _go2vec_sc(idx, emb_weights, n_ch=n_ch, b_per_w=b_per_w)
    return out.reshape(B_, H_, D)


# speedup vs baseline: 1.0445x; 1.0445x over previous
import functools

import jax
import jax.numpy as jnp
from jax import lax
from jax.experimental import pallas as pl
from jax.experimental.pallas import tpu as pltpu
from jax.experimental.pallas import tpu_sc as plsc

D = 32
NC = 2
NS = 16
NW = NC * NS
CH = 128
NBUF = 4


@functools.partial(jax.jit, static_argnames=("n_ch", "b_per_w"))
def _go2vec_sc(idx, table, *, n_ch, b_per_w):
    B = NW * b_per_w
    mesh = plsc.VectorSubcoreMesh(core_axis_name="c", subcore_axis_name="s")

    @functools.partial(
        pl.kernel,
        mesh=mesh,
        out_type=jax.ShapeDtypeStruct((B, D), jnp.float32),
        scratch_types=[
            pltpu.VMEM((n_ch, CH), jnp.int32),
            pltpu.VMEM((NBUF, CH, D), jnp.float32),
            pltpu.SemaphoreType.DMA,
            pltpu.SemaphoreType.DMA,
        ],
        compiler_params=pltpu.CompilerParams(use_tc_tiling_on_sc=False),
    )
    def k(idx_hbm, table_hbm, out_hbm, idx_v, rows_v, gsem, osem):
        wid = lax.axis_index("s") * NC + lax.axis_index("c")
        base = wid * b_per_w

        pltpu.sync_copy(idx_hbm.at[wid], idx_v)

        def g_copy(j, p):
            return pltpu.make_async_copy(
                table_hbm.at[idx_v.at[j]], rows_v.at[p], gsem)

        def o_copy(j, p):
            return pltpu.make_async_copy(
                rows_v.at[p], out_hbm.at[pl.ds(base + j * CH, CH)], osem)

        g_copy(0, 0).start()

        def body(j, carry):
            p = lax.rem(j, NBUF)
            g_copy(j, p).wait()
            o_copy(j, p).start()

            @pl.when(j < n_ch - 1)
            def _():
                pn = lax.rem(j + 1, NBUF)

                @pl.when(j >= NBUF - 1)
                def _():
                    o_copy(j - (NBUF - 1), pn).wait()

                g_copy(j + 1, pn).start()

            return carry

        lax.fori_loop(0, n_ch, body, 0)

        for b in range(NBUF):
            o_copy(0, b).wait()

    return k(idx, table)


def kernel(go, emb_weights):
    B_, H_ = go.shape
    B = B_ * H_
    b_per_w = B // NW
    n_ch = b_per_w // CH
    idx = go.reshape(NW, n_ch, CH).astype(jnp.int32)
    out = _go2vec_sc(idx, emb_weights, n_ch=n_ch, b_per_w=b_per_w)
    return out.reshape(B_, H_, D)


# speedup vs baseline: 1.5013x; 1.4373x over previous
import functools

import jax
import jax.numpy as jnp
from jax import lax
from jax.experimental import pallas as pl
from jax.experimental.pallas import tpu as pltpu
from jax.experimental.pallas import tpu_sc as plsc

D = 32
NC = 2
NS = 16
NW = NC * NS
CH = 128
BLK = 512
NCH = BLK // CH


@functools.partial(jax.jit, static_argnames=("hist", "batch"))
def _go2vec_sc(idx, table, *, hist, batch):
    mesh = plsc.VectorSubcoreMesh(core_axis_name="c", subcore_axis_name="s")
    assert hist % 2 == 0 and batch == NW * BLK

    @functools.partial(
        pl.kernel,
        mesh=mesh,
        out_type=jax.ShapeDtypeStruct((hist, D, batch), jnp.float32),
        scratch_types=[
            pltpu.VMEM((hist, NCH, CH), jnp.int32),
            pltpu.VMEM((2, BLK, D), jnp.float32),
            pltpu.VMEM((2, D, BLK), jnp.float32),
            pltpu.SemaphoreType.DMA,
            pltpu.SemaphoreType.DMA,
        ],
        compiler_params=pltpu.CompilerParams(
            use_tc_tiling_on_sc=False, needs_layout_passes=False),
    )
    def k(idx_hbm, table_hbm, out_hbm, idx_v, rows_v, trans_v, gsem, osem):
        wid = lax.axis_index("s") * NC + lax.axis_index("c")
        base = wid * BLK

        pltpu.sync_copy(idx_hbm.at[:, pl.ds(wid * NCH, NCH)], idx_v)

        def g_copy(h, p, c):
            return pltpu.make_async_copy(
                table_hbm.at[idx_v.at[h, c]],
                rows_v.at[p, pl.ds(c * CH, CH)], gsem)

        def o_copy(h, p):
            return pltpu.make_async_copy(
                trans_v.at[p], out_hbm.at[h, :, pl.ds(base, BLK)], osem)

        def transpose(p):
            rows = rows_v.at[p]
            lanes = lax.iota(jnp.int32, 16)

            def tr_body(g, carry):
                bvec = g * 16 + lanes
                for d in range(D):
                    dvec = jnp.full((16,), d, jnp.int32)
                    trans_v[p, d, pl.ds(g * 16, 16)] = plsc.load_gather(
                        rows, [bvec, dvec])
                return carry

            lax.fori_loop(0, BLK // 16, tr_body, 0)

        def step(h, p, hh):
            for c in range(NCH):
                g_copy(h, p, c).wait()

            if p == 0:
                for c in range(NCH):
                    g_copy(h + 1, 1 - p, c).start()
            else:
                @pl.when(hh < hist // 2 - 1)
                def _():
                    for c in range(NCH):
                        g_copy(h + 1, 1 - p, c).start()

            @pl.when(hh >= 1)
            def _():
                o_copy(h - 2, p).wait()

            transpose(p)
            o_copy(h, p).start()

        for c in range(NCH):
            g_copy(0, 0, c).start()

        def pair(hh, carry):
            step(2 * hh, 0, hh)
            step(2 * hh + 1, 1, hh)
            return carry

        lax.fori_loop(0, hist // 2, pair, 0)

        o_copy(hist - 2, 0).wait()
        o_copy(hist - 1, 1).wait()

    return k(idx, table)


def kernel(go, emb_weights):
    batch, hist = go.shape
    idx = go.T.reshape(hist, batch // CH, CH).astype(jnp.int32)
    out = _go2vec_sc(idx, emb_weights, hist=hist, batch=batch)
    return out.transpose(2, 0, 1)


# speedup vs baseline: 1.8540x; 1.2349x over previous
import functools

import jax
import jax.numpy as jnp
from jax import lax
from jax.experimental import pallas as pl
from jax.experimental.pallas import tpu as pltpu
from jax.experimental.pallas import tpu_sc as plsc

D = 32
NC = 2
NS = 16
NW = NC * NS
CH = 128
BLK = 512
NCH = BLK // CH


@functools.partial(jax.jit, static_argnames=("hist", "batch"))
def _go2vec_sc(idx, table, *, hist, batch):
    mesh = plsc.VectorSubcoreMesh(core_axis_name="c", subcore_axis_name="s")
    assert hist % 2 == 0 and batch == NW * BLK

    @functools.partial(
        pl.kernel,
        mesh=mesh,
        out_type=jax.ShapeDtypeStruct((hist, D, batch), jnp.float32),
        scratch_types=[
            pltpu.VMEM((hist, NCH, CH), jnp.int32),
            pltpu.VMEM((2, BLK, D), jnp.float32),
            pltpu.VMEM((2, D, BLK), jnp.float32),
            pltpu.SemaphoreType.DMA,
            pltpu.SemaphoreType.DMA,
        ],
        compiler_params=pltpu.CompilerParams(
            use_tc_tiling_on_sc=False, needs_layout_passes=False),
    )
    def k(idx_hbm, table_hbm, out_hbm, idx_v, rows_v, trans_v, gsem, osem):
        wid = lax.axis_index("s") * NC + lax.axis_index("c")
        base = wid * BLK

        pltpu.sync_copy(idx_hbm.at[:, pl.ds(wid * NCH, NCH)], idx_v)

        def g_copy(h, p, c):
            return pltpu.make_async_copy(
                table_hbm.at[idx_v.at[h, c]],
                rows_v.at[p, pl.ds(c * CH, CH)], gsem)

        def o_copy(h, p):
            return pltpu.make_async_copy(
                trans_v.at[p], out_hbm.at[h, :, pl.ds(base, BLK)], osem)

        def transpose(p):
            rows = rows_v.at[p]
            lanes = lax.iota(jnp.int32, 16)

            @plsc.parallel_loop(0, BLK // 16, unroll=2)
            def tr_body(g):
                bvec = g * 16 + lanes
                for d in range(D):
                    dvec = jnp.full((16,), d, jnp.int32)
                    trans_v[p, d, pl.ds(g * 16, 16)] = plsc.load_gather(
                        rows, [bvec, dvec])

        def step(h, p, hh):
            for c in range(NCH):
                g_copy(h, p, c).wait()

            if p == 0:
                for c in range(NCH):
                    g_copy(h + 1, 1 - p, c).start()
            else:
                @pl.when(hh < hist // 2 - 1)
                def _():
                    for c in range(NCH):
                        g_copy(h + 1, 1 - p, c).start()

            @pl.when(hh >= 1)
            def _():
                o_copy(h - 2, p).wait()

            transpose(p)
            o_copy(h, p).start()

        for c in range(NCH):
            g_copy(0, 0, c).start()

        def pair(hh, carry):
            step(2 * hh, 0, hh)
            step(2 * hh + 1, 1, hh)
            return carry

        lax.fori_loop(0, hist // 2, pair, 0)

        o_copy(hist - 2, 0).wait()
        o_copy(hist - 1, 1).wait()

    return k(idx, table)


def kernel(go, emb_weights):
    batch, hist = go.shape
    idx = go.T.reshape(hist, batch // CH, CH).astype(jnp.int32)
    out = _go2vec_sc(idx, emb_weights, hist=hist, batch=batch)
    return out.transpose(2, 0, 1)


# speedup vs baseline: 2.0037x; 1.0808x over previous
import functools

import jax
import jax.numpy as jnp
from jax import lax
from jax.experimental import pallas as pl
from jax.experimental.pallas import tpu as pltpu
from jax.experimental.pallas import tpu_sc as plsc

D = 32
NC = 2
NS = 16
NW = NC * NS
CH = 128
BLK = 512
NCH = BLK // CH
TC = 512


def _table_transpose(table_t, tail_rm):
    V = table_t.shape[1]
    n_chunks = V // TC
    v_main = n_chunks * TC
    tail = V - v_main
    mesh = plsc.VectorSubcoreMesh(core_axis_name="c", subcore_axis_name="s")

    @functools.partial(
        pl.kernel,
        mesh=mesh,
        out_type=jax.ShapeDtypeStruct((V // 4, 4 * D), jnp.float32),
        scratch_types=[
            pltpu.VMEM((2, D, TC), jnp.float32),
            pltpu.VMEM((2, TC // 4, 4 * D), jnp.float32),
            pltpu.SemaphoreType.DMA,
            pltpu.SemaphoreType.DMA,
        ],
        compiler_params=pltpu.CompilerParams(
            use_tc_tiling_on_sc=True, needs_layout_passes=False),
    )
    def k(tab_hbm, tail_hbm, out_hbm, slab_v, rows_v, isem, osem):
        wid = lax.axis_index("s") * NC + lax.axis_index("c")
        lanes = lax.iota(jnp.int32, 16)

        def i_copy(chunk, p):
            return pltpu.make_async_copy(
                tab_hbm.at[:, pl.ds(chunk * TC, TC)], slab_v.at[p], isem)

        def o_copy(chunk, p):
            return pltpu.make_async_copy(
                rows_v.at[p], out_hbm.at[pl.ds(chunk * (TC // 4), TC // 4)],
                osem)

        def transpose(p, nrows):
            @plsc.parallel_loop(0, nrows, unroll=2)
            def tr_body(r):
                for q in range(4 * D // 16):
                    dvec = (q % 2) * 16 + lanes
                    cvec = jnp.zeros((16,), jnp.int32) + (4 * r + q // 2)
                    rows_v[p, r, pl.ds(q * 16, 16)] = plsc.load_gather(
                        slab_v.at[p], [dvec, cvec])

        n_mine = (n_chunks - 1 - wid) // NW + 1
        i_copy(wid, 0).start()

        def body(g, carry):
            chunk = wid + g * NW
            p = lax.rem(g, 2)
            i_copy(chunk, p).wait()

            @pl.when(g < n_mine - 1)
            def _():
                i_copy(chunk + NW, 1 - p).start()

            @pl.when(g >= 2)
            def _():
                o_copy(0, p).wait()

            transpose(p, TC // 4)
            o_copy(chunk, p).start()
            return carry

        lax.fori_loop(0, n_mine, body, 0)

        o_copy(0, 0).wait()
        o_copy(0, 1).wait()

        if tail:
            @pl.when(wid == 0)
            def _():
                pltpu.sync_copy(tail_hbm, rows_v.at[0, pl.ds(0, tail // 4)])
                pltpu.sync_copy(rows_v.at[0, pl.ds(0, tail // 4)],
                                out_hbm.at[pl.ds(v_main // 4, tail // 4)])

    return k(table_t, tail_rm)


def _go2vec_sc(idx, table, hist, batch):
    mesh = plsc.VectorSubcoreMesh(core_axis_name="c", subcore_axis_name="s")
    assert hist % 2 == 0 and batch == NW * BLK

    @functools.partial(
        pl.kernel,
        mesh=mesh,
        out_type=jax.ShapeDtypeStruct((hist, D, batch), jnp.float32),
        scratch_types=[
            pltpu.VMEM((hist, NCH, CH), jnp.int32),
            pltpu.VMEM((2, BLK, D), jnp.float32),
            pltpu.VMEM((2, D, BLK), jnp.float32),
            pltpu.SemaphoreType.DMA,
            pltpu.SemaphoreType.DMA,
        ],
        compiler_params=pltpu.CompilerParams(
            use_tc_tiling_on_sc=False, needs_layout_passes=False),
    )
    def k(idx_hbm, table_hbm, out_hbm, idx_v, rows_v, trans_v, gsem, osem):
        wid = lax.axis_index("s") * NC + lax.axis_index("c")
        base = wid * BLK

        pltpu.sync_copy(idx_hbm.at[:, pl.ds(wid * NCH, NCH)], idx_v)

        def g_copy(h, p, c):
            return pltpu.make_async_copy(
                table_hbm.at[idx_v.at[h, c]],
                rows_v.at[p, pl.ds(c * CH, CH)], gsem)

        def o_copy(h, p):
            return pltpu.make_async_copy(
                trans_v.at[p], out_hbm.at[h, :, pl.ds(base, BLK)], osem)

        def transpose(p):
            rows = rows_v.at[p]
            lanes = lax.iota(jnp.int32, 16)

            @plsc.parallel_loop(0, BLK // 16, unroll=2)
            def tr_body(g):
                bvec = g * 16 + lanes
                for d in range(D):
                    dvec = jnp.full((16,), d, jnp.int32)
                    trans_v[p, d, pl.ds(g * 16, 16)] = plsc.load_gather(
                        rows, [bvec, dvec])

        def step(h, p, hh):
            for c in range(NCH):
                g_copy(h, p, c).wait()

            if p == 0:
                for c in range(NCH):
                    g_copy(h + 1, 1 - p, c).start()
            else:
                @pl.when(hh < hist // 2 - 1)
                def _():
                    for c in range(NCH):
                        g_copy(h + 1, 1 - p, c).start()

            @pl.when(hh >= 1)
            def _():
                o_copy(h - 2, p).wait()

            transpose(p)
            o_copy(h, p).start()

        for c in range(NCH):
            g_copy(0, 0, c).start()

        def pair(hh, carry):
            step(2 * hh, 0, hh)
            step(2 * hh + 1, 1, hh)
            return carry

        lax.fori_loop(0, hist // 2, pair, 0)

        o_copy(hist - 2, 0).wait()
        o_copy(hist - 1, 1).wait()

    return k(idx, table)


@functools.partial(jax.jit, static_argnames=("hist", "batch", "vocab"))
def _impl(go, emb_weights, *, hist, batch, vocab):
    idx = go.T.reshape(hist, batch // CH, CH).astype(jnp.int32)
    v_main = (vocab // TC) * TC
    tail_rm = emb_weights[v_main:].reshape((vocab - v_main) // 4, 4 * D)
    t128 = _table_transpose(emb_weights.T, tail_rm)
    tbl_rm = t128.reshape(vocab, D)
    out = _go2vec_sc(idx, tbl_rm, hist, batch)
    return out.transpose(2, 0, 1)


def kernel(go, emb_weights):
    batch, hist = go.shape
    return _impl(go, emb_weights, hist=hist, batch=batch,
                 vocab=emb_weights.shape[0])


# speedup vs baseline: 2.0148x; 1.0055x over previous
import functools

import jax
import jax.numpy as jnp
from jax import lax
from jax.experimental import pallas as pl
from jax.experimental.pallas import tpu as pltpu
from jax.experimental.pallas import tpu_sc as plsc

D = 32
NC = 2
NS = 16
NW = NC * NS
CH = 128
BLK = 512
NCH = BLK // CH
TC = 512


def _table_transpose(table_t, tail_rm):
    V = table_t.shape[1]
    n_chunks = V // TC
    v_main = n_chunks * TC
    tail = V - v_main
    mesh = plsc.VectorSubcoreMesh(core_axis_name="c", subcore_axis_name="s")

    @functools.partial(
        pl.kernel,
        mesh=mesh,
        out_type=jax.ShapeDtypeStruct((V // 4, 4 * D), jnp.float32),
        scratch_types=[
            pltpu.VMEM((2, 4 * D, CH), jnp.float32),
            pltpu.VMEM((2, TC // 4, 4 * D), jnp.float32),
            pltpu.SemaphoreType.DMA,
            pltpu.SemaphoreType.DMA,
        ],
        compiler_params=pltpu.CompilerParams(
            use_tc_tiling_on_sc=True, needs_layout_passes=False),
    )
    def k(tab_hbm, tail_hbm, out_hbm, slab_v, rows_v, isem, osem):
        wid = lax.axis_index("s") * NC + lax.axis_index("c")
        lanes = lax.iota(jnp.int32, 16)

        def i_copy(chunk, p, j):
            return pltpu.make_async_copy(
                tab_hbm.at[:, pl.ds(chunk * TC + j * CH, CH)],
                slab_v.at[p, pl.ds(j * D, D)], isem)

        def o_copy(chunk, p):
            return pltpu.make_async_copy(
                rows_v.at[p], out_hbm.at[pl.ds(chunk * (TC // 4), TC // 4)],
                osem)

        def transpose(p, nrows):
            @plsc.parallel_loop(0, nrows, unroll=2)
            def tr_body(r):
                panel = r // (CH // 4)
                rm = lax.rem(r, CH // 4)
                for q in range(4 * D // 16):
                    rowvec = panel * D + (q % 2) * 16 + lanes
                    cvec = jnp.zeros((16,), jnp.int32) + (4 * rm + q // 2)
                    rows_v[p, r, pl.ds(q * 16, 16)] = plsc.load_gather(
                        slab_v.at[p], [rowvec, cvec])

        n_mine = (n_chunks - 1 - wid) // NW + 1
        for j in range(TC // CH):
            i_copy(wid, 0, j).start()

        def body(g, carry):
            chunk = wid + g * NW
            p = lax.rem(g, 2)
            for j in range(TC // CH):
                i_copy(chunk, p, j).wait()

            @pl.when(g < n_mine - 1)
            def _():
                for j in range(TC // CH):
                    i_copy(chunk + NW, 1 - p, j).start()

            @pl.when(g >= 2)
            def _():
                o_copy(0, p).wait()

            transpose(p, TC // 4)
            o_copy(chunk, p).start()
            return carry

        lax.fori_loop(0, n_mine, body, 0)

        o_copy(0, 0).wait()
        o_copy(0, 1).wait()

        if tail:
            @pl.when(wid == 0)
            def _():
                pltpu.sync_copy(tail_hbm, rows_v.at[0, pl.ds(0, tail // 4)])
                pltpu.sync_copy(rows_v.at[0, pl.ds(0, tail // 4)],
                                out_hbm.at[pl.ds(v_main // 4, tail // 4)])

    return k(table_t, tail_rm)


def _go2vec_sc(idx, table, hist, batch):
    mesh = plsc.VectorSubcoreMesh(core_axis_name="c", subcore_axis_name="s")
    assert hist % 2 == 0 and batch == NW * BLK

    @functools.partial(
        pl.kernel,
        mesh=mesh,
        out_type=jax.ShapeDtypeStruct((hist, D, batch), jnp.float32),
        scratch_types=[
            pltpu.VMEM((hist, NCH, CH), jnp.int32),
            pltpu.VMEM((2, BLK, D), jnp.float32),
            pltpu.VMEM((2, D, BLK), jnp.float32),
            pltpu.SemaphoreType.DMA,
            pltpu.SemaphoreType.DMA,
        ],
        compiler_params=pltpu.CompilerParams(
            use_tc_tiling_on_sc=False, needs_layout_passes=False),
    )
    def k(idx_hbm, table_hbm, out_hbm, idx_v, rows_v, trans_v, gsem, osem):
        wid = lax.axis_index("s") * NC + lax.axis_index("c")
        base = wid * BLK

        pltpu.sync_copy(idx_hbm.at[:, pl.ds(wid * NCH, NCH)], idx_v)

        def g_copy(h, p, c):
            return pltpu.make_async_copy(
                table_hbm.at[idx_v.at[h, c]],
                rows_v.at[p, pl.ds(c * CH, CH)], gsem)

        def o_copy(h, p):
            return pltpu.make_async_copy(
                trans_v.at[p], out_hbm.at[h, :, pl.ds(base, BLK)], osem)

        def transpose(p):
            rows = rows_v.at[p]
            lanes = lax.iota(jnp.int32, 16)

            @plsc.parallel_loop(0, BLK // 16, unroll=2)
            def tr_body(g):
                bvec = g * 16 + lanes
                for d in range(D):
                    dvec = jnp.full((16,), d, jnp.int32)
                    trans_v[p, d, pl.ds(g * 16, 16)] = plsc.load_gather(
                        rows, [bvec, dvec])

        def step(h, p, hh):
            for c in range(NCH):
                g_copy(h, p, c).wait()

            if p == 0:
                for c in range(NCH):
                    g_copy(h + 1, 1 - p, c).start()
            else:
                @pl.when(hh < hist // 2 - 1)
                def _():
                    for c in range(NCH):
                        g_copy(h + 1, 1 - p, c).start()

            @pl.when(hh >= 1)
            def _():
                o_copy(h - 2, p).wait()

            transpose(p)
            o_copy(h, p).start()

        for c in range(NCH):
            g_copy(0, 0, c).start()

        def pair(hh, carry):
            step(2 * hh, 0, hh)
            step(2 * hh + 1, 1, hh)
            return carry

        lax.fori_loop(0, hist // 2, pair, 0)

        o_copy(hist - 2, 0).wait()
        o_copy(hist - 1, 1).wait()

    return k(idx, table)


@functools.partial(jax.jit, static_argnames=("hist", "batch", "vocab"))
def _impl(go, emb_weights, *, hist, batch, vocab):
    idx = go.T.reshape(hist, batch // CH, CH).astype(jnp.int32)
    v_main = (vocab // TC) * TC
    tail_rm = emb_weights[v_main:].reshape((vocab - v_main) // 4, 4 * D)
    t128 = _table_transpose(emb_weights.T, tail_rm)
    tbl_rm = t128.reshape(vocab, D)
    out = _go2vec_sc(idx, tbl_rm, hist, batch)
    return out.transpose(2, 0, 1)


def kernel(go, emb_weights):
    batch, hist = go.shape
    return _impl(go, emb_weights, hist=hist, batch=batch,
                 vocab=emb_weights.shape[0])
